# baseline (speedup 1.0000x reference)
import functools

import jax
import jax.numpy as jnp
import numpy as np
from jax import lax
from jax.experimental import pallas as pl
from jax.experimental.pallas import tpu as pltpu
from jax.experimental.pallas import tpu_sc as plsc

_TOPK = 12
_LANES = 16


def _scores_body(x_ref, wkq_ref, a_ref, *, group, n, d):
    xall = x_ref[...].reshape(group * n, x_ref.shape[2])
    kqt = lax.dot_general(wkq_ref[...], xall, (((0,), (1,)), ((), ())),
                          preferred_element_type=jnp.float32)
    for g in range(group):
        xkt = kqt[:d, g * n:(g + 1) * n]
        xqt = kqt[d:, g * n:(g + 1) * n]
        s = lax.dot_general(xkt, xqt, (((0,), (0,)), ((), ())),
                            preferred_element_type=jnp.float32)
        m = jnp.max(s, axis=-1, keepdims=True)
        e = jnp.exp(s - m)
        p = e / jnp.sum(e, axis=-1, keepdims=True)
        a_ref[g, 0, :] = jnp.sum(p, axis=0)


def _topk_body(rows_per_worker, n, a_hbm, out_hbm, row_v, out_v):
    wid = lax.axis_index("s") * 2 + lax.axis_index("c")
    nchunk = n // _LANES
    iota = lax.iota(jnp.int32, _LANES)
    neg = jnp.float32(-jnp.inf)

    def do_row(r, _):
        row = wid * rows_per_worker + r
        pltpu.sync_copy(a_hbm.at[row], row_v)

        def chunk_merge(c, kv):
            keys, vals = kv
            v = row_v[pl.ds(c * _LANES, _LANES)]
            gidx = c * _LANES + iota
            vs, vi = plsc.sort_key_val(v, gidx, descending=False)
            take = keys >= vs
            mk = jnp.where(take, keys, vs)
            mv = jnp.where(take, vals, vi)
            ks, vs2 = plsc.sort_key_val(mk, mv, descending=True)
            return (ks, vs2)

        keys, vals = lax.fori_loop(
            0, nchunk, chunk_merge,
            (jnp.full((_LANES,), neg, jnp.float32),
             jnp.zeros((_LANES,), jnp.int32)))
        out_v[pl.ds(r * _TOPK, _LANES)] = vals
        return 0

    lax.fori_loop(0, rows_per_worker, do_row, 0)
    npack = rows_per_worker * _TOPK
    pltpu.sync_copy(out_v.at[pl.ds(0, npack)],
                    out_hbm.at[pl.ds(wid * npack, npack)])


def kernel(x, flat):
    N, T, n, d_in = x.shape
    d = flat.shape[0] // (2 * d_in)
    B = N * T
    wk = flat[: d_in * d].reshape(d_in, d)
    wq = flat[d_in * d:].reshape(d_in, d)
    xf = x.reshape(B, n, d_in)
    scale = np.float32(1.0 / np.sqrt(np.float32(d_in)))

    group = 8
    nsplit = 2
    bs = B // nsplit
    wkq = jnp.concatenate([wk, wq * scale], axis=1)

    info = plsc.get_sparse_core_info()
    nworkers = info.num_cores * info.num_subcores
    rows_per_worker = bs // nworkers
    mesh = plsc.VectorSubcoreMesh(core_axis_name="c", subcore_axis_name="s")

    pieces = []
    for sp in range(nsplit):
        xs = xf[sp * bs:(sp + 1) * bs]
        a = pl.pallas_call(
            functools.partial(_scores_body, group=group, n=n, d=d),
            grid=(bs // group,),
            in_specs=[
                pl.BlockSpec((group, n, d_in), lambda b: (b, 0, 0)),
                pl.BlockSpec((d_in, 2 * d), lambda b: (0, 0)),
            ],
            out_specs=pl.BlockSpec((group, 1, n), lambda b: (b, 0, 0)),
            out_shape=jax.ShapeDtypeStruct((bs, 1, n), jnp.float32),
        )(xs, wkq)
        a = a.reshape(bs, n)

        pieces.append(pl.kernel(
            functools.partial(_topk_body, rows_per_worker, n),
            out_type=jax.ShapeDtypeStruct((bs * _TOPK,), jnp.int32),
            mesh=mesh,
            compiler_params=pltpu.CompilerParams(needs_layout_passes=False),
            scratch_types=[
                pltpu.VMEM((n,), jnp.float32),
                pltpu.VMEM((rows_per_worker * _TOPK + _LANES,), jnp.int32),
            ],
        )(a))

    topk = jnp.concatenate(pieces)
    return topk.reshape(N, T, _TOPK, 1)


# speedup vs baseline: 1.5933x; 1.5933x over previous
import functools

import jax
import jax.numpy as jnp
import numpy as np
from jax import lax
from jax.experimental import pallas as pl
from jax.experimental.pallas import tpu as pltpu
from jax.experimental.pallas import tpu_sc as plsc

_TOPK = 12
_LANES = 16


def _scores_body(x_ref, wk_ref, wq_ref, a_ref, *, group, n, d, scale):
    wkq = jnp.concatenate([wk_ref[...], wq_ref[...] * scale], axis=1)
    xall = x_ref[...].reshape(group * n, x_ref.shape[2])
    kqt = lax.dot_general(wkq, xall, (((0,), (1,)), ((), ())),
                          preferred_element_type=jnp.float32)
    for g in range(group):
        xkt = kqt[:d, g * n:(g + 1) * n]
        xqt = kqt[d:, g * n:(g + 1) * n]
        s = lax.dot_general(xkt, xqt, (((0,), (0,)), ((), ())),
                            preferred_element_type=jnp.float32)
        m = jnp.max(s, axis=-1, keepdims=True)
        e = jnp.exp(s - m)
        p = e / jnp.sum(e, axis=-1, keepdims=True)
        a_ref[g, 0, :] = jnp.sum(p, axis=0)


def _topk_body(rows_per_worker, n, a_hbm, out_hbm, row_v, out_v):
    wid = lax.axis_index("s") * 2 + lax.axis_index("c")
    nchunk = n // _LANES
    iota = lax.iota(jnp.int32, _LANES)
    neg = jnp.float32(-jnp.inf)

    def do_row(r, _):
        row = wid * rows_per_worker + r
        pltpu.sync_copy(a_hbm.at[row], row_v)

        def chunk_merge(c, kv):
            keys, vals = kv
            v = row_v[pl.ds(c * _LANES, _LANES)]
            gidx = c * _LANES + iota
            vs, vi = plsc.sort_key_val(v, gidx, descending=False)
            take = keys >= vs
            mk = jnp.where(take, keys, vs)
            mv = jnp.where(take, vals, vi)
            ks, vs2 = plsc.sort_key_val(mk, mv, descending=True)
            return (ks, vs2)

        keys, vals = lax.fori_loop(
            0, nchunk, chunk_merge,
            (jnp.full((_LANES,), neg, jnp.float32),
             jnp.zeros((_LANES,), jnp.int32)))
        out_v[pl.ds(r * _TOPK, _LANES)] = vals
        return 0

    lax.fori_loop(0, rows_per_worker, do_row, 0)
    npack = rows_per_worker * _TOPK
    pltpu.sync_copy(out_v.at[pl.ds(0, npack)],
                    out_hbm.at[pl.ds(wid * npack, npack)])


def kernel(x, flat):
    N, T, n, d_in = x.shape
    d = flat.shape[0] // (2 * d_in)
    B = N * T
    wk = flat[: d_in * d].reshape(d_in, d)
    wq = flat[d_in * d:].reshape(d_in, d)
    xf = x.reshape(B, n, d_in)
    scale = np.float32(1.0 / np.sqrt(np.float32(d_in)))

    group = 8
    a = pl.pallas_call(
        functools.partial(_scores_body, group=group, n=n, d=d, scale=scale),
        grid=(B // group,),
        in_specs=[
            pl.BlockSpec((group, n, d_in), lambda b: (b, 0, 0)),
            pl.BlockSpec((d_in, d), lambda b: (0, 0)),
            pl.BlockSpec((d_in, d), lambda b: (0, 0)),
        ],
        out_specs=pl.BlockSpec((group, 1, n), lambda b: (b, 0, 0)),
        out_shape=jax.ShapeDtypeStruct((B, 1, n), jnp.float32),
    )(xf, wk, wq)
    a = a.reshape(B, n)

    info = plsc.get_sparse_core_info()
    nworkers = info.num_cores * info.num_subcores
    rows_per_worker = B // nworkers
    mesh = plsc.VectorSubcoreMesh(core_axis_name="c", subcore_axis_name="s")

    topk = pl.kernel(
        functools.partial(_topk_body, rows_per_worker, n),
        out_type=jax.ShapeDtypeStruct((B * _TOPK,), jnp.int32),
        mesh=mesh,
        compiler_params=pltpu.CompilerParams(needs_layout_passes=False),
        scratch_types=[
            pltpu.VMEM((n,), jnp.float32),
            pltpu.VMEM((rows_per_worker * _TOPK + _LANES,), jnp.int32),
        ],
    )(a)

    return topk.reshape(N, T, _TOPK, 1)


# speedup vs baseline: 1.6622x; 1.0432x over previous
import functools

import jax
import jax.numpy as jnp
import numpy as np
from jax import lax
from jax.experimental import pallas as pl
from jax.experimental.pallas import tpu as pltpu
from jax.experimental.pallas import tpu_sc as plsc

_TOPK = 12
_LANES = 16


def _scores_body(x_ref, wkq_ref, a_ref, *, group, n, d):
    xall = x_ref[...].reshape(group * n, x_ref.shape[2])
    kqt = lax.dot_general(wkq_ref[...], xall, (((0,), (1,)), ((), ())),
                          preferred_element_type=jnp.float32)
    for g in range(group):
        xkt = kqt[:d, g * n:(g + 1) * n]
        xqt = kqt[d:, g * n:(g + 1) * n]
        s = lax.dot_general(xkt, xqt, (((0,), (0,)), ((), ())),
                            preferred_element_type=jnp.float32)
        m = jnp.max(s, axis=-1, keepdims=True)
        e = jnp.exp(s - m)
        p = e / jnp.sum(e, axis=-1, keepdims=True)
        a_ref[g, :] = jnp.sum(p, axis=0)


def _topk_body(rows_per_worker, n, a_hbm, out_hbm, row_v, out_v):
    wid = lax.axis_index("s") * 2 + lax.axis_index("c")
    nchunk = n // _LANES
    iota = lax.iota(jnp.int32, _LANES)
    neg = jnp.float32(-jnp.inf)

    def do_row(r, _):
        row = wid * rows_per_worker + r
        pltpu.sync_copy(a_hbm.at[row], row_v)

        def chunk_merge(c, kv):
            keys, vals = kv
            v = row_v[pl.ds(c * _LANES, _LANES)]
            gidx = c * _LANES + iota
            vs, vi = plsc.sort_key_val(v, gidx, descending=False)
            take = keys >= vs
            mk = jnp.where(take, keys, vs)
            mv = jnp.where(take, vals, vi)
            ks, vs2 = plsc.sort_key_val(mk, mv, descending=True)
            return (ks, vs2)

        keys, vals = lax.fori_loop(
            0, nchunk, chunk_merge,
            (jnp.full((_LANES,), neg, jnp.float32),
             jnp.zeros((_LANES,), jnp.int32)))
        out_v[pl.ds(r * _TOPK, _LANES)] = vals
        return 0

    lax.fori_loop(0, rows_per_worker, do_row, 0)
    npack = rows_per_worker * _TOPK
    pltpu.sync_copy(out_v.at[pl.ds(0, npack)],
                    out_hbm.at[pl.ds(wid * npack, npack)])


def kernel(x, flat):
    N, T, n, d_in = x.shape
    d = flat.shape[0] // (2 * d_in)
    B = N * T
    wk = flat[: d_in * d].reshape(d_in, d)
    wq = flat[d_in * d:].reshape(d_in, d)
    xf = x.reshape(B, n, d_in)
    scale = np.float32(1.0 / np.sqrt(np.float32(d_in)))

    group = 8
    wkq = jnp.concatenate([wk, wq * scale], axis=1)
    a = pl.pallas_call(
        functools.partial(_scores_body, group=group, n=n, d=d),
        grid=(B // group,),
        in_specs=[
            pl.BlockSpec((group, n, d_in), lambda b: (b, 0, 0)),
            pl.BlockSpec((d_in, 2 * d), lambda b: (0, 0)),
        ],
        out_specs=pl.BlockSpec((group, n), lambda b: (b, 0)),
        out_shape=jax.ShapeDtypeStruct((B, n), jnp.float32),
    )(xf, wkq)

    info = plsc.get_sparse_core_info()
    nworkers = info.num_cores * info.num_subcores
    rows_per_worker = B // nworkers
    mesh = plsc.VectorSubcoreMesh(core_axis_name="c", subcore_axis_name="s")

    topk = pl.kernel(
        functools.partial(_topk_body, rows_per_worker, n),
        out_type=jax.ShapeDtypeStruct((B * _TOPK,), jnp.int32),
        mesh=mesh,
        compiler_params=pltpu.CompilerParams(needs_layout_passes=False),
        scratch_types=[
            pltpu.VMEM((n,), jnp.float32),
            pltpu.VMEM((rows_per_worker * _TOPK + _LANES,), jnp.int32),
        ],
    )(a)

    return topk.reshape(N, T, _TOPK, 1)


# speedup vs baseline: 1.6917x; 1.0177x over previous
import functools

import jax
import jax.numpy as jnp
import numpy as np
from jax import lax
from jax.experimental import pallas as pl
from jax.experimental.pallas import tpu as pltpu
from jax.experimental.pallas import tpu_sc as plsc

_TOPK = 12
_LANES = 16


def _scores_body(x_ref, wkq_ref, a_ref, *, group, n, d):
    xall = x_ref[...].reshape(group * n, x_ref.shape[2])
    kqt = lax.dot_general(wkq_ref[...], xall, (((0,), (1,)), ((), ())),
                          preferred_element_type=jnp.float32)
    for g in range(group):
        xkt = kqt[:d, g * n:(g + 1) * n]
        xqt = kqt[d:, g * n:(g + 1) * n]
        s = lax.dot_general(xkt, xqt, (((0,), (0,)), ((), ())),
                            preferred_element_type=jnp.float32)
        m = jnp.max(s, axis=-1, keepdims=True)
        e = jnp.exp(s - m)
        p = e / jnp.sum(e, axis=-1, keepdims=True)
        a_ref[g, :] = jnp.sum(p, axis=0)


def _topk_body(rows_per_worker, n, a_hbm, out_hbm, row_a, row_b, out_v,
               sem_a, sem_b):
    wid = lax.axis_index("s") * 2 + lax.axis_index("c")
    base = wid * rows_per_worker
    nchunk = n // _LANES
    iota = lax.iota(jnp.int32, _LANES)
    neg = jnp.float32(-jnp.inf)
    bufs = [row_a, row_b]
    sems = [sem_a, sem_b]

    copies = [None, None]
    copies[0] = pltpu.make_async_copy(a_hbm.at[base], row_a, sem_a)
    copies[0].start()
    for r in range(rows_per_worker):
        if r + 1 < rows_per_worker:
            copies[(r + 1) % 2] = pltpu.make_async_copy(
                a_hbm.at[base + r + 1], bufs[(r + 1) % 2], sems[(r + 1) % 2])
            copies[(r + 1) % 2].start()
        copies[r % 2].wait()
        row_v = bufs[r % 2]

        def chunk_merge(c, kv, row_v=row_v):
            keys, vals = kv
            v = row_v[pl.ds(c * _LANES, _LANES)]
            gidx = c * _LANES + iota
            vs, vi = plsc.sort_key_val(v, gidx, descending=False)
            take = keys >= vs
            mk = jnp.where(take, keys, vs)
            mv = jnp.where(take, vals, vi)
            ks, vs2 = plsc.sort_key_val(mk, mv, descending=True)
            return (ks, vs2)

        keys, vals = lax.fori_loop(
            0, nchunk, chunk_merge,
            (jnp.full((_LANES,), neg, jnp.float32),
             jnp.zeros((_LANES,), jnp.int32)))
        out_v[pl.ds(r * _TOPK, _LANES)] = vals

    npack = rows_per_worker * _TOPK
    pltpu.sync_copy(out_v.at[pl.ds(0, npack)],
                    out_hbm.at[pl.ds(wid * npack, npack)])


def kernel(x, flat):
    N, T, n, d_in = x.shape
    d = flat.shape[0] // (2 * d_in)
    B = N * T
    wk = flat[: d_in * d].reshape(d_in, d)
    wq = flat[d_in * d:].reshape(d_in, d)
    xf = x.reshape(B, n, d_in)
    scale = np.float32(1.0 / np.sqrt(np.float32(d_in)))

    group = 8
    wkq = jnp.concatenate([wk, wq * scale], axis=1)
    a = pl.pallas_call(
        functools.partial(_scores_body, group=group, n=n, d=d),
        grid=(B // group,),
        in_specs=[
            pl.BlockSpec((group, n, d_in), lambda b: (b, 0, 0)),
            pl.BlockSpec((d_in, 2 * d), lambda b: (0, 0)),
        ],
        out_specs=pl.BlockSpec((group, n), lambda b: (b, 0)),
        out_shape=jax.ShapeDtypeStruct((B, n), jnp.float32),
    )(xf, wkq)

    info = plsc.get_sparse_core_info()
    nworkers = info.num_cores * info.num_subcores
    rows_per_worker = B // nworkers
    mesh = plsc.VectorSubcoreMesh(core_axis_name="c", subcore_axis_name="s")

    topk = pl.kernel(
        functools.partial(_topk_body, rows_per_worker, n),
        out_type=jax.ShapeDtypeStruct((B * _TOPK,), jnp.int32),
        mesh=mesh,
        compiler_params=pltpu.CompilerParams(needs_layout_passes=False),
        scratch_types=[
            pltpu.VMEM((n,), jnp.float32),
            pltpu.VMEM((n,), jnp.float32),
            pltpu.VMEM((rows_per_worker * _TOPK + _LANES,), jnp.int32),
            pltpu.SemaphoreType.DMA,
            pltpu.SemaphoreType.DMA,
        ],
    )(a)

    return topk.reshape(N, T, _TOPK, 1)
